# baseline (speedup 1.0000x reference)
import functools

import jax
import jax.numpy as jnp
from jax import lax
from jax.experimental import pallas as pl
from jax.experimental.pallas import tpu as pltpu
from jax.experimental.pallas import tpu_sc as plsc

D = 64
NC, NS = 2, 16
NW = NC * NS
CHUNK = 128
NBUF = 5
L = 16


def _pair_table(emb, W, b):
    def body(emb_ref, w_ref, b_ref, out_ref):
        t = lax.dot_general(
            emb_ref[...], w_ref[...], (((1,), (1,)), ((), ())),
            preferred_element_type=jnp.float32) + b_ref[...]
        left = jnp.broadcast_to(t[:, None, :], (D, D, D))
        right = jnp.broadcast_to(t[None, :, :], (D, D, D))
        out_ref[...] = jnp.concatenate([left, right], axis=-1)

    return pl.pallas_call(
        body,
        out_shape=jax.ShapeDtypeStruct((D, D, 2 * D), jnp.float32),
    )(emb, W, b.reshape(1, D))


@functools.partial(jax.jit, static_argnames=("n_chunks",))
def _sc_gather(table2, packed, n_chunks):
    n_pairs_w = n_chunks * CHUNK
    n_rows = NW * n_pairs_w
    mesh = plsc.VectorSubcoreMesh(
        core_axis_name="c", subcore_axis_name="s",
        num_cores=NC, num_subcores=NS)

    @functools.partial(
        pl.kernel,
        out_type=jax.ShapeDtypeStruct((n_rows, 2 * D), jnp.float32),
        mesh=mesh,
        scratch_types=[
            pltpu.VMEM((n_pairs_w,), jnp.int32),
            pltpu.VMEM((NBUF, CHUNK, 2 * D), jnp.float32),
            pltpu.SemaphoreType.DMA((NBUF,)),
            pltpu.SemaphoreType.DMA((NBUF,)),
        ],
    )
    def k(table_hbm, pk_hbm, out_hbm, pidx_v, bufs, gsem, wsem):
        wid = lax.axis_index("s") * NC + lax.axis_index("c")
        base = wid * n_pairs_w
        pltpu.sync_copy(pk_hbm.at[pl.ds(base, n_pairs_w)], pidx_v)

        def pair_body(j, carry):
            s = pl.ds(j * L, L)
            v = pidx_v[s]
            pidx_v[s] = (v & 0xFFFF) * D + (v >> 16)
            return carry

        lax.fori_loop(0, n_pairs_w // L, pair_body, 0)

        def start_gather(ch, p):
            pltpu.async_copy(
                table_hbm.at[pidx_v.at[pl.ds(ch * CHUNK, CHUNK)]],
                bufs.at[p], gsem.at[p])

        def wait_gather(ch, p):
            pltpu.make_async_copy(
                table_hbm.at[pidx_v.at[pl.ds(ch * CHUNK, CHUNK)]],
                bufs.at[p], gsem.at[p]).wait()

        def out_slice(ch):
            return out_hbm.at[pl.ds(base + ch * CHUNK, CHUNK)]

        for p in range(NBUF):
            start_gather(p, p)

        def body(i, carry):
            for p in range(NBUF):
                ch = i * NBUF + p
                wait_gather(ch, p)
                pltpu.async_copy(bufs.at[p], out_slice(ch), wsem.at[p])
            for p in range(NBUF):
                ch = i * NBUF + p
                pltpu.make_async_copy(
                    bufs.at[p], out_slice(ch), wsem.at[p]).wait()
                start_gather((i + 1) * NBUF + p, p)
            return carry

        lax.fori_loop(0, n_chunks // NBUF - 1, body, 0)

        last = n_chunks - NBUF
        for p in range(NBUF):
            ch = last + p
            wait_gather(ch, p)
            pltpu.async_copy(bufs.at[p], out_slice(ch), wsem.at[p])
        for p in range(NBUF):
            ch = last + p
            pltpu.make_async_copy(
                bufs.at[p], out_slice(ch), wsem.at[p]).wait()

    return k(table2, packed)


NSPLIT = 4


def _unpair_n(rows_list, bsz, seq):
    BB = 32
    half = seq // 2
    hb = bsz // NSPLIT

    def body(*refs):
        out_ref = refs[-1]
        for j, r in enumerate(refs[:-1]):
            v = r[...]
            out_ref[j] = jnp.concatenate(
                [v[:, :D].reshape(BB, half, D),
                 v[:, D:].reshape(BB, half, D)], axis=1)

    return pl.pallas_call(
        body,
        grid=(hb // BB,),
        in_specs=[pl.BlockSpec((BB * half, 2 * D), lambda g: (g, 0))
                  for _ in range(NSPLIT)],
        out_specs=pl.BlockSpec(
            (NSPLIT, BB, seq, D), lambda g: (0, g, 0, 0)),
        out_shape=jax.ShapeDtypeStruct((NSPLIT, hb, seq, D), jnp.float32),
    )(*rows_list)


def kernel(x, emb, W, b):
    bsz, seq = x.shape
    n = bsz * seq
    half = seq // 2
    n_pairs = n // 2
    shard = n_pairs // NSPLIT
    n_chunks = shard // (NW * CHUNK)
    packed = lax.bitcast_convert_type(
        jnp.stack([x[:, :half], x[:, half:]], axis=-1).astype(jnp.int16),
        jnp.int32).reshape(n_pairs)
    table2 = _pair_table(emb, W, b).reshape(D * D, 2 * D)
    rows = [_sc_gather(table2, packed[j * shard:(j + 1) * shard], n_chunks)
            for j in range(NSPLIT)]
    out = _unpair_n(rows, bsz, seq)
    return out.reshape(bsz, seq, D)


# speedup vs baseline: 1.0183x; 1.0183x over previous
import functools

import jax
import jax.numpy as jnp
from jax import lax
from jax.experimental import pallas as pl
from jax.experimental.pallas import tpu as pltpu
from jax.experimental.pallas import tpu_sc as plsc

D = 64
NC, NS = 2, 16
NW = NC * NS
CHUNK = 128
NBUF = 5
L = 16


def _pair_table(emb, W, b):
    def body(emb_ref, w_ref, b_ref, out_ref):
        t = lax.dot_general(
            emb_ref[...], w_ref[...], (((1,), (1,)), ((), ())),
            preferred_element_type=jnp.float32) + b_ref[...]
        left = jnp.broadcast_to(t[:, None, :], (D, D, D))
        right = jnp.broadcast_to(t[None, :, :], (D, D, D))
        out_ref[...] = jnp.concatenate([left, right], axis=-1)

    return pl.pallas_call(
        body,
        out_shape=jax.ShapeDtypeStruct((D, D, 2 * D), jnp.float32),
    )(emb, W, b.reshape(1, D))


@functools.partial(jax.jit, static_argnames=("n_chunks",))
def _sc_gather(table2, packed, n_chunks):
    n_pairs_w = n_chunks * CHUNK
    n_rows = NW * n_pairs_w
    mesh = plsc.VectorSubcoreMesh(
        core_axis_name="c", subcore_axis_name="s",
        num_cores=NC, num_subcores=NS)

    @functools.partial(
        pl.kernel,
        out_type=jax.ShapeDtypeStruct((n_rows, 2 * D), jnp.float32),
        mesh=mesh,
        scratch_types=[
            pltpu.VMEM((n_pairs_w,), jnp.int32),
            pltpu.VMEM((NBUF, CHUNK, 2 * D), jnp.float32),
            pltpu.SemaphoreType.DMA((NBUF,)),
            pltpu.SemaphoreType.DMA((NBUF,)),
        ],
    )
    def k(table_hbm, pk_hbm, out_hbm, pidx_v, bufs, gsem, wsem):
        wid = lax.axis_index("s") * NC + lax.axis_index("c")
        base = wid * n_pairs_w
        pltpu.sync_copy(pk_hbm.at[pl.ds(base, n_pairs_w)], pidx_v)

        def pair_body(j, carry):
            s = pl.ds(j * L, L)
            v = pidx_v[s]
            pidx_v[s] = (v & 0xFFFF) * D + (v >> 16)
            return carry

        lax.fori_loop(0, n_pairs_w // L, pair_body, 0)

        def start_gather(ch, p):
            pltpu.async_copy(
                table_hbm.at[pidx_v.at[pl.ds(ch * CHUNK, CHUNK)]],
                bufs.at[p], gsem.at[p])

        def wait_gather(ch, p):
            pltpu.make_async_copy(
                table_hbm.at[pidx_v.at[pl.ds(ch * CHUNK, CHUNK)]],
                bufs.at[p], gsem.at[p]).wait()

        def out_slice(ch):
            return out_hbm.at[pl.ds(base + ch * CHUNK, CHUNK)]

        for p in range(NBUF):
            start_gather(p, p)

        def body(i, carry):
            for p in range(NBUF):
                ch = i * NBUF + p
                wait_gather(ch, p)
                pltpu.async_copy(bufs.at[p], out_slice(ch), wsem.at[p])
            for p in range(NBUF):
                ch = i * NBUF + p
                pltpu.make_async_copy(
                    bufs.at[p], out_slice(ch), wsem.at[p]).wait()
                start_gather((i + 1) * NBUF + p, p)
            return carry

        lax.fori_loop(0, n_chunks // NBUF - 1, body, 0)

        last = n_chunks - NBUF
        for p in range(NBUF):
            ch = last + p
            wait_gather(ch, p)
            pltpu.async_copy(bufs.at[p], out_slice(ch), wsem.at[p])
        for p in range(NBUF):
            ch = last + p
            pltpu.make_async_copy(
                bufs.at[p], out_slice(ch), wsem.at[p]).wait()

    return k(table2, packed)


def _unpair_shard(rows, j, bsz, seq, prev=None):
    BB = 32
    half = seq // 2
    hb = bsz // 2

    def body(r_ref, *rest):
        out_ref = rest[-1]
        v = r_ref[...]
        out_ref[0] = jnp.concatenate(
            [v[:, :D].reshape(BB, half, D),
             v[:, D:].reshape(BB, half, D)], axis=1)

    in_specs = [pl.BlockSpec((BB * half, 2 * D), lambda g: (g, 0))]
    args = [rows]
    aliases = {}
    if prev is not None:
        in_specs.append(pl.BlockSpec(memory_space=pl.ANY))
        args.append(prev)
        aliases = {1: 0}

    return pl.pallas_call(
        body,
        grid=(hb // BB,),
        in_specs=in_specs,
        out_specs=pl.BlockSpec(
            (1, BB, seq, D), lambda g: (j, g, 0, 0)),
        out_shape=jax.ShapeDtypeStruct((2, hb, seq, D), jnp.float32),
        input_output_aliases=aliases,
    )(*args)


def kernel(x, emb, W, b):
    bsz, seq = x.shape
    n = bsz * seq
    half = seq // 2
    n_chunks = n // (2 * NW * 2 * CHUNK)
    packed = lax.bitcast_convert_type(
        jnp.stack([x[:, :half], x[:, half:]], axis=-1).astype(jnp.int16),
        jnp.int32).reshape(n // 2)
    table2 = _pair_table(emb, W, b).reshape(D * D, 2 * D)
    rows_a = _sc_gather(table2, packed[: n // 4], n_chunks)
    rows_b = _sc_gather(table2, packed[n // 4:], n_chunks)
    u1 = _unpair_shard(rows_a, 0, bsz, seq)
    u2 = _unpair_shard(rows_b, 1, bsz, seq, prev=u1)
    return u2.reshape(bsz, seq, D)
